# baseline (speedup 1.0000x reference)
import functools

import jax
import jax.numpy as jnp
from jax import lax
from jax.experimental import pallas as pl
from jax.experimental.pallas import tpu as pltpu
from jax.experimental.pallas import tpu_sc as plsc

_L = 16
_NC = 2
_NS = 16
_NW = _NC * _NS
_B = 16384
_D = 64
_BPW = _B // _NW
_CHUNK = 128
_NCH = _BPW // _CHUNK


def _sc_body(uidx_hbm, iidx_hbm, utab_hbm, itab_hbm, out_hbm,
             uidx_v, iidx_v, urows_v, irows_v, out_v, sem):
    wid = lax.axis_index("s") * _NC + lax.axis_index("c")
    base = wid * _BPW

    pltpu.sync_copy(uidx_hbm.at[pl.ds(wid * _NCH, _NCH)], uidx_v)
    pltpu.sync_copy(iidx_hbm.at[pl.ds(wid * _NCH, _NCH)], iidx_v)

    copies = []
    for c in range(_NCH):
        copies.append(pltpu.async_copy(
            utab_hbm.at[uidx_v.at[c]],
            urows_v.at[pl.ds(c * _CHUNK, _CHUNK)], sem))
        copies.append(pltpu.async_copy(
            itab_hbm.at[iidx_v.at[c]],
            irows_v.at[pl.ds(c * _CHUNK, _CHUNK)], sem))
    for cp in copies:
        cp.wait()

    lane = lax.iota(jnp.int32, _L)
    rots = [jnp.bitwise_and(lane + (1 << t), _L - 1) for t in range(4)]

    def group_body(g, _):
        r0 = g * _L
        accv = jnp.zeros((_L,), jnp.float32)
        for j in range(_L):
            r = r0 + j
            p = urows_v[r, pl.ds(0, _L)] * irows_v[r, pl.ds(0, _L)]
            for k in range(1, _D // _L):
                p = p + (urows_v[r, pl.ds(k * _L, _L)]
                         * irows_v[r, pl.ds(k * _L, _L)])
            for t in range(4):
                p = p + jnp.take(p, rots[t], axis=0)
            accv = jnp.where(lane == j, p, accv)
        out_v[pl.ds(r0, _L)] = accv
        return _

    lax.fori_loop(0, _BPW // _L, group_body, 0)

    pltpu.sync_copy(out_v, out_hbm.at[pl.ds(base, _BPW)])


@jax.jit
def _run(uidx2, iidx2, user_table, item_table):
    mesh = plsc.VectorSubcoreMesh(core_axis_name="c", subcore_axis_name="s")
    f = pl.kernel(
        _sc_body,
        mesh=mesh,
        out_type=jax.ShapeDtypeStruct((_B,), jnp.float32),
        compiler_params=pltpu.CompilerParams(use_tc_tiling_on_sc=False),
        scratch_types=[
            pltpu.VMEM((_NCH, _CHUNK), jnp.int32),
            pltpu.VMEM((_NCH, _CHUNK), jnp.int32),
            pltpu.VMEM((_BPW, _D), jnp.float32),
            pltpu.VMEM((_BPW, _D), jnp.float32),
            pltpu.VMEM((_BPW,), jnp.float32),
            pltpu.SemaphoreType.DMA,
        ],
    )
    return f(uidx2, iidx2, user_table, item_table)


def kernel(user_indices, item_indices, user_table, item_table):
    uidx2 = user_indices.astype(jnp.int32).reshape(_NW * _NCH, _CHUNK)
    iidx2 = item_indices.astype(jnp.int32).reshape(_NW * _NCH, _CHUNK)
    return _run(uidx2, iidx2, user_table, item_table)
